# Initial kernel scaffold; baseline (speedup 1.0000x reference)
import jax
import jax.numpy as jnp
from jax.experimental import pallas as pl


def kernel(edge_features, node_features, edge_indices, W1, b1, W2, b2):
    raise NotImplementedError("write your pallas kernel here")


# speedup vs baseline: 2.0233x; 2.0233x over previous
import functools

import jax
import jax.numpy as jnp
from jax import lax
from jax.experimental import pallas as pl
from jax.experimental.pallas import tpu as pltpu
from jax.experimental.pallas import tpu_sc as plsc

N_NODES = 10000
N_EDGES = 320000
D_IN = 128
D_OUT = 128
D_EDGE = 16
_EPS = 1e-5

_NC = 2
_NS = 16
_NW = _NC * _NS
_EPW = N_EDGES // _NW
_CHUNK = 80
_NCHUNK = _EPW // _CHUNK


def _gelu(x):
    return 0.5 * x * (1.0 + lax.erf(x * 0.7071067811865476))


def _node_kernel(x_ref, w1_ref, b1_ref, w2b_ref, b2_ref, out_ref):
    y = jnp.dot(x_ref[...], w1_ref[...], preferred_element_type=jnp.float32)
    y = y + b1_ref[...]
    mu = jnp.mean(y)
    var = jnp.mean((y - mu) ** 2)
    h = (y - mu) * lax.rsqrt(var + _EPS)
    h = _gelu(h)
    out_ref[...] = (
        jnp.dot(h, w2b_ref[...], preferred_element_type=jnp.float32)
        + b2_ref[...]
    )


def _node_phase(x, W1, b1, W2b, b2):
    return pl.pallas_call(
        _node_kernel,
        out_shape=jax.ShapeDtypeStruct((N_NODES, D_OUT), jnp.float32),
    )(x, W1, b1, W2b, b2)


def _sc_gather_body(h2_hbm, src_hbm, r_hbm, idx_v, rows_v, sem):
    wid = lax.axis_index("s") * _NC + lax.axis_index("c")
    base = wid * _EPW
    for i in range(_NCHUNK):
        off = base + i * _CHUNK
        pltpu.sync_copy(src_hbm.at[pl.ds(off, _CHUNK)], idx_v)
        pltpu.async_copy(h2_hbm.at[idx_v], rows_v, sem).wait()
        pltpu.sync_copy(rows_v, r_hbm.at[pl.ds(off, _CHUNK)])


def _sc_gather(h2, src):
    mesh = plsc.VectorSubcoreMesh(core_axis_name="c", subcore_axis_name="s")
    kern = functools.partial(
        pl.kernel,
        mesh=mesh,
        out_type=jax.ShapeDtypeStruct((N_EDGES, D_OUT), jnp.float32),
        scratch_types=[
            pltpu.VMEM((_CHUNK,), jnp.int32),
            pltpu.VMEM((_CHUNK, D_OUT), jnp.float32),
            pltpu.SemaphoreType.DMA,
        ],
    )(_sc_gather_body)
    return kern(h2, src)


_E_BLK = 1000
_NB = N_EDGES // _E_BLK
_N_ELEMS = float(N_EDGES * D_OUT)


def _stats_kernel(ef_ref, r_ref, w2a_ref, out_ref, acc_ref):
    j = pl.program_id(0)

    t = jnp.dot(ef_ref[...], w2a_ref[...], preferred_element_type=jnp.float32)
    t = t + r_ref[...]

    @pl.when(j == 0)
    def _init():
        acc_ref[0] = 0.0
        acc_ref[1] = 0.0

    acc_ref[0] += jnp.sum(t)
    acc_ref[1] += jnp.sum(t * t)

    @pl.when(j == _NB - 1)
    def _finalize():
        mu = acc_ref[0] / _N_ELEMS
        var = acc_ref[1] / _N_ELEMS - mu * mu
        out_ref[0] = mu
        out_ref[1] = lax.rsqrt(var + _EPS)


def _edge_stats(ef, r, W2a):
    return pl.pallas_call(
        _stats_kernel,
        grid=(_NB,),
        in_specs=[
            pl.BlockSpec((_E_BLK, D_EDGE), lambda j: (j, 0)),
            pl.BlockSpec((_E_BLK, D_OUT), lambda j: (j, 0)),
            pl.BlockSpec((D_EDGE, D_OUT), lambda j: (0, 0)),
        ],
        out_specs=pl.BlockSpec(memory_space=pltpu.SMEM),
        out_shape=jax.ShapeDtypeStruct((2,), jnp.float32),
        scratch_shapes=[pltpu.SMEM((2,), jnp.float32)],
    )(ef, r, W2a)


def _apply_kernel(stats_ref, ef_ref, r_ref, w2a_ref, out_ref):
    t = jnp.dot(ef_ref[...], w2a_ref[...], preferred_element_type=jnp.float32)
    t = t + r_ref[...]
    out_ref[...] = _gelu((t - stats_ref[0]) * stats_ref[1])


def _edge_apply(ef, r, W2a, stats):
    return pl.pallas_call(
        _apply_kernel,
        grid=(_NB,),
        in_specs=[
            pl.BlockSpec(memory_space=pltpu.SMEM),
            pl.BlockSpec((_E_BLK, D_EDGE), lambda j: (j, 0)),
            pl.BlockSpec((_E_BLK, D_OUT), lambda j: (j, 0)),
            pl.BlockSpec((D_EDGE, D_OUT), lambda j: (0, 0)),
        ],
        out_specs=pl.BlockSpec((_E_BLK, D_OUT), lambda j: (j, 0)),
        out_shape=jax.ShapeDtypeStruct((N_EDGES, D_OUT), jnp.float32),
    )(stats, ef, r, W2a)


_N_HALF = N_NODES // _NC
_ACC_ROWS = 5120
_ZROWS = _ACC_ROWS // _NS
_ZCHUNK = 80
_TRASH = _N_HALF
_EPS_SC = N_EDGES // _NS
_NCHUNK_SC = _EPS_SC // _CHUNK
_L = 16


def _sc_scatter_body(u_hbm, dst_hbm, on1_hbm, zc1_hbm, pu_hbm, pc_hbm,
                     idx_v, idx2_v, rows_v, ones1_v, z1_v, acc_u, acc_c,
                     sem):
    c = lax.axis_index("c")
    s = lax.axis_index("s")

    zf = jnp.zeros((_L,), jnp.float32)

    def _fill0(i, carry):
        for k in range(D_OUT // _L):
            rows_v[i, pl.ds(k * _L, _L)] = zf
        return carry

    lax.fori_loop(0, _CHUNK, _fill0, 0)
    pltpu.sync_copy(on1_hbm, ones1_v)
    for z in range(_ZROWS // _ZCHUNK):
        pltpu.sync_copy(rows_v.at[pl.ds(0, _ZCHUNK)],
                        acc_u.at[pl.ds(s * _ZROWS + z * _ZCHUNK, _ZCHUNK)])
    pltpu.sync_copy(zc1_hbm, z1_v)
    pltpu.sync_copy(z1_v, acc_c.at[pl.ds(s * _ZROWS, _ZROWS)])
    plsc.subcore_barrier()

    base = c * _N_HALF

    def _chunk(i, carry):
        off = s * _EPS_SC + i * _CHUNK
        pltpu.sync_copy(dst_hbm.at[pl.ds(off, _CHUNK)], idx_v)
        pltpu.sync_copy(u_hbm.at[pl.ds(off, _CHUNK)], rows_v)
        for k in range(_CHUNK // _L):
            v = idx_v[pl.ds(k * _L, _L)] - base
            ok = (v >= 0) & (v < _N_HALF)
            idx2_v[pl.ds(k * _L, _L)] = jnp.where(ok, v, _TRASH)
        pltpu.sync_copy(rows_v, acc_u.at[idx2_v], add=True)
        pltpu.sync_copy(ones1_v, acc_c.at[idx2_v], add=True)
        return carry

    lax.fori_loop(0, _NCHUNK_SC, _chunk, 0)

    plsc.subcore_barrier()

    @pl.when(s == 0)
    def _writeback():
        pltpu.sync_copy(acc_u.at[pl.ds(0, _N_HALF)], pu_hbm.at[c])
        pltpu.sync_copy(acc_c, pc_hbm.at[pl.ds(c * _ACC_ROWS, _ACC_ROWS)])


def _sc_scatter(u, dst):
    on1 = jnp.ones((_CHUNK,), jnp.float32)
    zc1 = jnp.zeros((_ZROWS,), jnp.float32)
    mesh = plsc.VectorSubcoreMesh(core_axis_name="c", subcore_axis_name="s")
    kern = functools.partial(
        pl.kernel,
        mesh=mesh,
        out_type=(
            jax.ShapeDtypeStruct((_NC, _N_HALF, D_OUT), jnp.float32),
            jax.ShapeDtypeStruct((_NC * _ACC_ROWS,), jnp.float32),
        ),
        scratch_types=[
            pltpu.VMEM((_CHUNK,), jnp.int32),
            pltpu.VMEM((_CHUNK,), jnp.int32),
            pltpu.VMEM((_CHUNK, D_OUT), jnp.float32),
            pltpu.VMEM((_CHUNK,), jnp.float32),
            pltpu.VMEM((_ZROWS,), jnp.float32),
            pltpu.VMEM_SHARED((_ACC_ROWS, D_OUT), jnp.float32),
            pltpu.VMEM_SHARED((_ACC_ROWS,), jnp.float32),
            pltpu.SemaphoreType.DMA,
        ],
    )(_sc_scatter_body)
    return kern(u, dst, on1, zc1)


def _combine_kernel(pu_ref, cnt_ref, out_ref):
    s = jnp.concatenate([pu_ref[0], pu_ref[1]], axis=0)
    cnt = cnt_ref[...]
    out_ref[...] = jnp.where(cnt > 0.0, s / jnp.maximum(cnt, 1.0), 0.0)


def _combine(pu, pc):
    cnt = jnp.concatenate(
        [pc[0:_N_HALF], pc[_ACC_ROWS:_ACC_ROWS + _N_HALF]]).reshape(
            N_NODES, 1)
    return pl.pallas_call(
        _combine_kernel,
        out_shape=jax.ShapeDtypeStruct((N_NODES, D_OUT), jnp.float32),
    )(pu, cnt)


def kernel(edge_features, node_features, edge_indices, W1, b1, W2, b2):
    src = edge_indices[:, 0].astype(jnp.int32)
    dst = edge_indices[:, 1].astype(jnp.int32)
    W2a = W2[:D_EDGE]
    W2b = W2[D_EDGE:]
    h2 = _node_phase(node_features, W1, b1.reshape(1, D_OUT), W2b,
                     b2.reshape(1, D_OUT))
    r = _sc_gather(h2, src)
    stats = _edge_stats(edge_features, r, W2a)
    u = _edge_apply(edge_features, r, W2a, stats)
    pu, pc = _sc_scatter(u, dst)
    return _combine(pu, pc)


# speedup vs baseline: 2.5332x; 1.2520x over previous
import functools

import jax
import jax.numpy as jnp
from jax import lax
from jax.experimental import pallas as pl
from jax.experimental.pallas import tpu as pltpu
from jax.experimental.pallas import tpu_sc as plsc

N_NODES = 10000
N_EDGES = 320000
D_IN = 128
D_OUT = 128
D_EDGE = 16
_EPS = 1e-5

_NC = 2
_NS = 16
_NW = _NC * _NS
_EPW = N_EDGES // _NW
_CHUNK = 80
_NCHUNK = _EPW // _CHUNK


def _gelu(x):
    return 0.5 * x * (1.0 + lax.erf(x * 0.7071067811865476))


def _node_kernel(x_ref, w1_ref, b1_ref, w2b_ref, b2_ref, out_ref):
    y = jnp.dot(x_ref[...], w1_ref[...], preferred_element_type=jnp.float32)
    y = y + b1_ref[...]
    mu = jnp.mean(y)
    var = jnp.mean((y - mu) ** 2)
    h = (y - mu) * lax.rsqrt(var + _EPS)
    h = _gelu(h)
    out_ref[...] = (
        jnp.dot(h, w2b_ref[...], preferred_element_type=jnp.float32)
        + b2_ref[...]
    )


def _node_phase(x, W1, b1, W2b, b2):
    return pl.pallas_call(
        _node_kernel,
        out_shape=jax.ShapeDtypeStruct((N_NODES, D_OUT), jnp.float32),
    )(x, W1, b1, W2b, b2)


_GB = 400
_GSUB = _GB // _CHUNK
_GITER = _EPW // _GB


def _sc_gather_body(h2_hbm, src_hbm, r_hbm, idx_v, rows_v, sem, semw):
    wid = lax.axis_index("s") * _NC + lax.axis_index("c")
    base = wid * _EPW

    def _iter(i, carry):
        off = base + i * _GB
        pltpu.sync_copy(src_hbm.at[pl.ds(off, _GB)], idx_v)
        cps = [
            pltpu.async_copy(
                h2_hbm.at[idx_v.at[pl.ds(k * _CHUNK, _CHUNK)]],
                rows_v.at[pl.ds(k * _CHUNK, _CHUNK)], sem)
            for k in range(_GSUB)
        ]
        for cp in cps:
            cp.wait()
        pltpu.async_copy(rows_v, r_hbm.at[pl.ds(off, _GB)], semw).wait()
        return carry

    lax.fori_loop(0, _GITER, _iter, 0)


def _sc_gather(h2, src):
    mesh = plsc.VectorSubcoreMesh(core_axis_name="c", subcore_axis_name="s")
    kern = functools.partial(
        pl.kernel,
        mesh=mesh,
        out_type=jax.ShapeDtypeStruct((N_EDGES, D_OUT), jnp.float32),
        scratch_types=[
            pltpu.VMEM((_GB,), jnp.int32),
            pltpu.VMEM((_GB, D_OUT), jnp.float32),
            pltpu.SemaphoreType.DMA,
            pltpu.SemaphoreType.DMA,
        ],
    )(_sc_gather_body)
    return kern(h2, src)


_E_BLK = 1000
_NB = N_EDGES // _E_BLK
_N_ELEMS = float(N_EDGES * D_OUT)


def _stats_kernel(ef_ref, r_ref, w2a_ref, out_ref, acc_ref):
    j = pl.program_id(0)

    t = jnp.dot(ef_ref[...], w2a_ref[...], preferred_element_type=jnp.float32)
    t = t + r_ref[...]

    @pl.when(j == 0)
    def _init():
        acc_ref[0] = 0.0
        acc_ref[1] = 0.0

    acc_ref[0] += jnp.sum(t)
    acc_ref[1] += jnp.sum(t * t)

    @pl.when(j == _NB - 1)
    def _finalize():
        mu = acc_ref[0] / _N_ELEMS
        var = acc_ref[1] / _N_ELEMS - mu * mu
        out_ref[0] = mu
        out_ref[1] = lax.rsqrt(var + _EPS)


def _edge_stats(ef, r, W2a):
    return pl.pallas_call(
        _stats_kernel,
        grid=(_NB,),
        in_specs=[
            pl.BlockSpec((_E_BLK, D_EDGE), lambda j: (j, 0)),
            pl.BlockSpec((_E_BLK, D_OUT), lambda j: (j, 0)),
            pl.BlockSpec((D_EDGE, D_OUT), lambda j: (0, 0)),
        ],
        out_specs=pl.BlockSpec(memory_space=pltpu.SMEM),
        out_shape=jax.ShapeDtypeStruct((2,), jnp.float32),
        scratch_shapes=[pltpu.SMEM((2,), jnp.float32)],
    )(ef, r, W2a)


def _apply_kernel(stats_ref, ef_ref, r_ref, w2a_ref, out_ref):
    t = jnp.dot(ef_ref[...], w2a_ref[...], preferred_element_type=jnp.float32)
    t = t + r_ref[...]
    out_ref[...] = _gelu((t - stats_ref[0]) * stats_ref[1])


def _edge_apply(ef, r, W2a, stats):
    return pl.pallas_call(
        _apply_kernel,
        grid=(_NB,),
        in_specs=[
            pl.BlockSpec(memory_space=pltpu.SMEM),
            pl.BlockSpec((_E_BLK, D_EDGE), lambda j: (j, 0)),
            pl.BlockSpec((_E_BLK, D_OUT), lambda j: (j, 0)),
            pl.BlockSpec((D_EDGE, D_OUT), lambda j: (0, 0)),
        ],
        out_specs=pl.BlockSpec((_E_BLK, D_OUT), lambda j: (j, 0)),
        out_shape=jax.ShapeDtypeStruct((N_EDGES, D_OUT), jnp.float32),
    )(stats, ef, r, W2a)


_N_HALF = N_NODES // _NC
_ACC_ROWS = 5120
_ZROWS = _ACC_ROWS // _NS
_ZCHUNK = 80
_TRASH = _N_HALF
_EPS_SC = N_EDGES // _NS
_NCHUNK_SC = _EPS_SC // _CHUNK
_L = 16


_SB = 320
_SSUB = _SB // _CHUNK
_SITER = _EPS_SC // _SB
_STAIL = _EPS_SC - _SITER * _SB


def _sc_scatter_body(u_hbm, dst_hbm, on1_hbm, zc1_hbm, pu_hbm, pc_hbm,
                     idx_v, i2a, i2b, i2c, i2d, rows_v, ones1_v, z1_v,
                     acc_u, acc_c, seml, sems):
    c = lax.axis_index("c")
    s = lax.axis_index("s")
    idx2 = [i2a, i2b, i2c, i2d]

    zf = jnp.zeros((_L,), jnp.float32)

    def _fill0(i, carry):
        for k in range(D_OUT // _L):
            rows_v[i, pl.ds(k * _L, _L)] = zf
        return carry

    lax.fori_loop(0, _SB, _fill0, 0)
    pltpu.sync_copy(on1_hbm, ones1_v)
    pltpu.sync_copy(rows_v, acc_u.at[pl.ds(s * _ZROWS, _ZROWS)])
    pltpu.sync_copy(zc1_hbm, z1_v)
    pltpu.sync_copy(z1_v, acc_c.at[pl.ds(s * _ZROWS, _ZROWS)])
    plsc.subcore_barrier()

    base = c * _N_HALF

    def _batch(off, nsub):
        n = nsub * _CHUNK
        cl = pltpu.async_copy(dst_hbm.at[pl.ds(off, n)],
                              idx_v.at[pl.ds(0, n)], seml)
        cu = pltpu.async_copy(u_hbm.at[pl.ds(off, n)],
                              rows_v.at[pl.ds(0, n)], seml)
        cl.wait()
        cu.wait()
        for k in range(nsub * (_CHUNK // _L)):
            v = idx_v[pl.ds(k * _L, _L)] - base
            ok = (v >= 0) & (v < _N_HALF)
            idx2[k // (_CHUNK // _L)][
                pl.ds((k % (_CHUNK // _L)) * _L, _L)] = jnp.where(
                    ok, v, _TRASH)
        cps = []
        for j in range(nsub):
            cps.append(pltpu.async_copy(
                rows_v.at[pl.ds(j * _CHUNK, _CHUNK)],
                acc_u.at[idx2[j]], sems, add=True))
            cps.append(pltpu.async_copy(
                ones1_v, acc_c.at[idx2[j]], sems, add=True))
        for cp in cps:
            cp.wait()

    def _iter(i, carry):
        _batch(s * _EPS_SC + i * _SB, _SSUB)
        return carry

    lax.fori_loop(0, _SITER, _iter, 0)
    _batch(s * _EPS_SC + _SITER * _SB, _STAIL // _CHUNK)

    plsc.subcore_barrier()

    @pl.when(s == 0)
    def _writeback():
        pltpu.sync_copy(acc_u.at[pl.ds(0, _N_HALF)], pu_hbm.at[c])
        pltpu.sync_copy(acc_c, pc_hbm.at[pl.ds(c * _ACC_ROWS, _ACC_ROWS)])


def _sc_scatter(u, dst):
    on1 = jnp.ones((_CHUNK,), jnp.float32)
    zc1 = jnp.zeros((_ZROWS,), jnp.float32)
    mesh = plsc.VectorSubcoreMesh(core_axis_name="c", subcore_axis_name="s")
    kern = functools.partial(
        pl.kernel,
        mesh=mesh,
        out_type=(
            jax.ShapeDtypeStruct((_NC, _N_HALF, D_OUT), jnp.float32),
            jax.ShapeDtypeStruct((_NC * _ACC_ROWS,), jnp.float32),
        ),
        scratch_types=[
            pltpu.VMEM((_SB,), jnp.int32),
            pltpu.VMEM((_CHUNK,), jnp.int32),
            pltpu.VMEM((_CHUNK,), jnp.int32),
            pltpu.VMEM((_CHUNK,), jnp.int32),
            pltpu.VMEM((_CHUNK,), jnp.int32),
            pltpu.VMEM((_SB, D_OUT), jnp.float32),
            pltpu.VMEM((_CHUNK,), jnp.float32),
            pltpu.VMEM((_ZROWS,), jnp.float32),
            pltpu.VMEM_SHARED((_ACC_ROWS, D_OUT), jnp.float32),
            pltpu.VMEM_SHARED((_ACC_ROWS,), jnp.float32),
            pltpu.SemaphoreType.DMA,
            pltpu.SemaphoreType.DMA,
        ],
    )(_sc_scatter_body)
    return kern(u, dst, on1, zc1)


def _combine_kernel(pu_ref, cnt_ref, out_ref):
    s = jnp.concatenate([pu_ref[0], pu_ref[1]], axis=0)
    cnt = cnt_ref[...]
    out_ref[...] = jnp.where(cnt > 0.0, s / jnp.maximum(cnt, 1.0), 0.0)


def _combine(pu, pc):
    cnt = jnp.concatenate(
        [pc[0:_N_HALF], pc[_ACC_ROWS:_ACC_ROWS + _N_HALF]]).reshape(
            N_NODES, 1)
    return pl.pallas_call(
        _combine_kernel,
        out_shape=jax.ShapeDtypeStruct((N_NODES, D_OUT), jnp.float32),
    )(pu, cnt)


def kernel(edge_features, node_features, edge_indices, W1, b1, W2, b2):
    src = edge_indices[:, 0].astype(jnp.int32)
    dst = edge_indices[:, 1].astype(jnp.int32)
    W2a = W2[:D_EDGE]
    W2b = W2[D_EDGE:]
    h2 = _node_phase(node_features, W1, b1.reshape(1, D_OUT), W2b,
                     b2.reshape(1, D_OUT))
    r = _sc_gather(h2, src)
    stats = _edge_stats(edge_features, r, W2a)
    u = _edge_apply(edge_features, r, W2a, stats)
    pu, pc = _sc_scatter(u, dst)
    return _combine(pu, pc)


# speedup vs baseline: 2.9852x; 1.1784x over previous
import functools

import jax
import jax.numpy as jnp
from jax import lax
from jax.experimental import pallas as pl
from jax.experimental.pallas import tpu as pltpu
from jax.experimental.pallas import tpu_sc as plsc

N_NODES = 10000
N_EDGES = 320000
D_IN = 128
D_OUT = 128
D_EDGE = 16
_EPS = 1e-5

_NC = 2
_NS = 16
_NW = _NC * _NS
_EPW = N_EDGES // _NW
_CHUNK = 80
_NCHUNK = _EPW // _CHUNK


def _gelu(x):
    return 0.5 * x * (1.0 + lax.erf(x * 0.7071067811865476))


def _node_kernel(x_ref, w1_ref, b1_ref, w2b_ref, b2_ref, out_ref):
    y = jnp.dot(x_ref[...], w1_ref[...], preferred_element_type=jnp.float32)
    y = y + b1_ref[...]
    mu = jnp.mean(y)
    var = jnp.mean((y - mu) ** 2)
    h = (y - mu) * lax.rsqrt(var + _EPS)
    h = _gelu(h)
    out_ref[...] = (
        jnp.dot(h, w2b_ref[...], preferred_element_type=jnp.float32)
        + b2_ref[...]
    )


def _node_phase(x, W1, b1, W2b, b2):
    return pl.pallas_call(
        _node_kernel,
        out_shape=jax.ShapeDtypeStruct((N_NODES, D_OUT), jnp.float32),
    )(x, W1, b1, W2b, b2)


_GB = 400
_GSUB = _GB // _CHUNK
_GITER = _EPW // _GB


def _sc_gather_body(h2_hbm, src_hbm, r_hbm, idx_v, rows_v, sem, semw):
    wid = lax.axis_index("s") * _NC + lax.axis_index("c")
    base = wid * _EPW

    def _iter(i, carry):
        off = base + i * _GB
        pltpu.sync_copy(src_hbm.at[pl.ds(off, _GB)], idx_v)
        cps = [
            pltpu.async_copy(
                h2_hbm.at[idx_v.at[pl.ds(k * _CHUNK, _CHUNK)]],
                rows_v.at[pl.ds(k * _CHUNK, _CHUNK)], sem)
            for k in range(_GSUB)
        ]
        for cp in cps:
            cp.wait()
        pltpu.async_copy(rows_v, r_hbm.at[pl.ds(off, _GB)], semw).wait()
        return carry

    lax.fori_loop(0, _GITER, _iter, 0)


def _sc_gather(h2, src):
    mesh = plsc.VectorSubcoreMesh(core_axis_name="c", subcore_axis_name="s")
    kern = functools.partial(
        pl.kernel,
        mesh=mesh,
        out_type=jax.ShapeDtypeStruct((N_EDGES, D_OUT), jnp.float32),
        scratch_types=[
            pltpu.VMEM((_GB,), jnp.int32),
            pltpu.VMEM((_GB, D_OUT), jnp.float32),
            pltpu.SemaphoreType.DMA,
            pltpu.SemaphoreType.DMA,
        ],
    )(_sc_gather_body)
    return kern(h2, src)


_E_BLK = 2000
_NB = N_EDGES // _E_BLK
_N_ELEMS = float(N_EDGES * D_OUT)


def _edge_kernel(ef_ref, r_ref, w2a_ref, out_ref, acc_ref):
    p = pl.program_id(0)
    j = pl.program_id(1)

    t = jnp.dot(ef_ref[...], w2a_ref[...], preferred_element_type=jnp.float32)
    t = t + r_ref[...]

    @pl.when(jnp.logical_and(p == 0, j == 0))
    def _init():
        acc_ref[0] = 0.0
        acc_ref[1] = 0.0

    @pl.when(p == 0)
    def _stats():
        acc_ref[0] += jnp.sum(t)
        acc_ref[1] += jnp.sum(t * t)

        @pl.when(j == _NB - 1)
        def _finalize():
            mu = acc_ref[0] / _N_ELEMS
            var = acc_ref[1] / _N_ELEMS - mu * mu
            acc_ref[2] = mu
            acc_ref[3] = lax.rsqrt(var + _EPS)

    @pl.when(p == 1)
    def _apply():
        out_ref[...] = _gelu((t - acc_ref[2]) * acc_ref[3])


def _edge_apply(ef, r, W2a):
    return pl.pallas_call(
        _edge_kernel,
        grid=(2, _NB),
        in_specs=[
            pl.BlockSpec((_E_BLK, D_EDGE), lambda p, j: (j, 0)),
            pl.BlockSpec((_E_BLK, D_OUT), lambda p, j: (j, 0)),
            pl.BlockSpec((D_EDGE, D_OUT), lambda p, j: (0, 0)),
        ],
        out_specs=pl.BlockSpec((_E_BLK, D_OUT), lambda p, j: (p * j, 0)),
        out_shape=jax.ShapeDtypeStruct((N_EDGES, D_OUT), jnp.float32),
        scratch_shapes=[pltpu.SMEM((4,), jnp.float32)],
    )(ef, r, W2a)


_N_HALF = N_NODES // _NC
_ACC_ROWS = 5120
_ZROWS = _ACC_ROWS // _NS
_ZCHUNK = 80
_TRASH = _N_HALF
_EPS_SC = N_EDGES // _NS
_NCHUNK_SC = _EPS_SC // _CHUNK
_L = 16


_SB = 320
_SSUB = _SB // _CHUNK
_SITER = _EPS_SC // _SB
_STAIL = _EPS_SC - _SITER * _SB


def _sc_scatter_body(u_hbm, dst_hbm, on1_hbm, zc1_hbm, pu_hbm, pc_hbm,
                     idx_v, i2a, i2b, i2c, i2d, rows_v, ones1_v, z1_v,
                     acc_u, acc_c, seml, sems):
    c = lax.axis_index("c")
    s = lax.axis_index("s")
    idx2 = [i2a, i2b, i2c, i2d]

    zf = jnp.zeros((_L,), jnp.float32)

    def _fill0(i, carry):
        for k in range(D_OUT // _L):
            rows_v[i, pl.ds(k * _L, _L)] = zf
        return carry

    lax.fori_loop(0, _SB, _fill0, 0)
    pltpu.sync_copy(on1_hbm, ones1_v)
    pltpu.sync_copy(rows_v, acc_u.at[pl.ds(s * _ZROWS, _ZROWS)])
    pltpu.sync_copy(zc1_hbm, z1_v)
    pltpu.sync_copy(z1_v, acc_c.at[pl.ds(s * _ZROWS, _ZROWS)])
    plsc.subcore_barrier()

    base = c * _N_HALF

    def _batch(off, nsub):
        n = nsub * _CHUNK
        cl = pltpu.async_copy(dst_hbm.at[pl.ds(off, n)],
                              idx_v.at[pl.ds(0, n)], seml)
        cu = pltpu.async_copy(u_hbm.at[pl.ds(off, n)],
                              rows_v.at[pl.ds(0, n)], seml)
        cl.wait()
        cu.wait()
        for k in range(nsub * (_CHUNK // _L)):
            v = idx_v[pl.ds(k * _L, _L)] - base
            ok = (v >= 0) & (v < _N_HALF)
            idx2[k // (_CHUNK // _L)][
                pl.ds((k % (_CHUNK // _L)) * _L, _L)] = jnp.where(
                    ok, v, _TRASH)
        cps = []
        for j in range(nsub):
            cps.append(pltpu.async_copy(
                rows_v.at[pl.ds(j * _CHUNK, _CHUNK)],
                acc_u.at[idx2[j]], sems, add=True))
            cps.append(pltpu.async_copy(
                ones1_v, acc_c.at[idx2[j]], sems, add=True))
        for cp in cps:
            cp.wait()

    def _iter(i, carry):
        _batch(s * _EPS_SC + i * _SB, _SSUB)
        return carry

    lax.fori_loop(0, _SITER, _iter, 0)
    _batch(s * _EPS_SC + _SITER * _SB, _STAIL // _CHUNK)

    plsc.subcore_barrier()

    @pl.when(s == 0)
    def _writeback():
        pltpu.sync_copy(acc_u.at[pl.ds(0, _N_HALF)], pu_hbm.at[c])
        pltpu.sync_copy(acc_c, pc_hbm.at[pl.ds(c * _ACC_ROWS, _ACC_ROWS)])


def _sc_scatter(u, dst):
    on1 = jnp.ones((_CHUNK,), jnp.float32)
    zc1 = jnp.zeros((_ZROWS,), jnp.float32)
    mesh = plsc.VectorSubcoreMesh(core_axis_name="c", subcore_axis_name="s")
    kern = functools.partial(
        pl.kernel,
        mesh=mesh,
        out_type=(
            jax.ShapeDtypeStruct((_NC, _N_HALF, D_OUT), jnp.float32),
            jax.ShapeDtypeStruct((_NC * _ACC_ROWS,), jnp.float32),
        ),
        scratch_types=[
            pltpu.VMEM((_SB,), jnp.int32),
            pltpu.VMEM((_CHUNK,), jnp.int32),
            pltpu.VMEM((_CHUNK,), jnp.int32),
            pltpu.VMEM((_CHUNK,), jnp.int32),
            pltpu.VMEM((_CHUNK,), jnp.int32),
            pltpu.VMEM((_SB, D_OUT), jnp.float32),
            pltpu.VMEM((_CHUNK,), jnp.float32),
            pltpu.VMEM((_ZROWS,), jnp.float32),
            pltpu.VMEM_SHARED((_ACC_ROWS, D_OUT), jnp.float32),
            pltpu.VMEM_SHARED((_ACC_ROWS,), jnp.float32),
            pltpu.SemaphoreType.DMA,
            pltpu.SemaphoreType.DMA,
        ],
    )(_sc_scatter_body)
    return kern(u, dst, on1, zc1)


def _combine_kernel(pu_ref, cnt_ref, out_ref):
    s = jnp.concatenate([pu_ref[0], pu_ref[1]], axis=0)
    cnt = cnt_ref[...]
    out_ref[...] = jnp.where(cnt > 0.0, s / jnp.maximum(cnt, 1.0), 0.0)


def _combine(pu, pc):
    cnt = jnp.concatenate(
        [pc[0:_N_HALF], pc[_ACC_ROWS:_ACC_ROWS + _N_HALF]]).reshape(
            N_NODES, 1)
    return pl.pallas_call(
        _combine_kernel,
        out_shape=jax.ShapeDtypeStruct((N_NODES, D_OUT), jnp.float32),
    )(pu, cnt)


def kernel(edge_features, node_features, edge_indices, W1, b1, W2, b2):
    src = edge_indices[:, 0].astype(jnp.int32)
    dst = edge_indices[:, 1].astype(jnp.int32)
    W2a = W2[:D_EDGE]
    W2b = W2[D_EDGE:]
    h2 = _node_phase(node_features, W1, b1.reshape(1, D_OUT), W2b,
                     b2.reshape(1, D_OUT))
    r = _sc_gather(h2, src)
    u = _edge_apply(edge_features, r, W2a)
    pu, pc = _sc_scatter(u, dst)
    return _combine(pu, pc)


# speedup vs baseline: 2.9878x; 1.0009x over previous
import functools

import jax
import jax.numpy as jnp
from jax import lax
from jax.experimental import pallas as pl
from jax.experimental.pallas import tpu as pltpu
from jax.experimental.pallas import tpu_sc as plsc

N_NODES = 10000
N_EDGES = 320000
D_IN = 128
D_OUT = 128
D_EDGE = 16
_EPS = 1e-5

_NC = 2
_NS = 16
_NW = _NC * _NS
_EPW = N_EDGES // _NW
_CHUNK = 80
_NCHUNK = _EPW // _CHUNK


def _gelu(x):
    return 0.5 * x * (1.0 + lax.erf(x * 0.7071067811865476))


def _node_kernel(x_ref, w1_ref, b1_ref, w2b_ref, b2_ref, out_ref):
    y = jnp.dot(x_ref[...], w1_ref[...], preferred_element_type=jnp.float32)
    y = y + b1_ref[...]
    mu = jnp.mean(y)
    var = jnp.mean((y - mu) ** 2)
    h = (y - mu) * lax.rsqrt(var + _EPS)
    h = _gelu(h)
    out_ref[...] = (
        jnp.dot(h, w2b_ref[...], preferred_element_type=jnp.float32)
        + b2_ref[...]
    )


def _node_phase(x, W1, b1, W2b, b2):
    return pl.pallas_call(
        _node_kernel,
        out_shape=jax.ShapeDtypeStruct((N_NODES, D_OUT), jnp.float32),
    )(x, W1, b1, W2b, b2)


_GB = 400
_GSUB = _GB // _CHUNK
_GITER = _EPW // _GB


def _sc_gather_body(h2_hbm, src_hbm, r_hbm,
                    idx_a, idx_b, rows_a, rows_b, sem, seml, semw):
    wid = lax.axis_index("s") * _NC + lax.axis_index("c")
    base = wid * _EPW
    idx = [idx_a, idx_b]
    rows = [rows_a, rows_b]
    wb = [None, None]
    lcp = [None, None]
    pltpu.sync_copy(src_hbm.at[pl.ds(base, _GB)], idx_a)
    for i in range(_GITER):
        b = i % 2
        nb = (i + 1) % 2
        if i + 1 < _GITER:
            lcp[nb] = pltpu.async_copy(
                src_hbm.at[pl.ds(base + (i + 1) * _GB, _GB)], idx[nb], seml)
        if wb[b] is not None:
            wb[b].wait()
        gs = [pltpu.async_copy(
                  h2_hbm.at[idx[b].at[pl.ds(k * _CHUNK, _CHUNK)]],
                  rows[b].at[pl.ds(k * _CHUNK, _CHUNK)], sem)
              for k in range(_GSUB)]
        for g in gs:
            g.wait()
        wb[b] = pltpu.async_copy(rows[b],
                                 r_hbm.at[pl.ds(base + i * _GB, _GB)], semw)
        if i + 1 < _GITER:
            lcp[nb].wait()
    for w in wb:
        if w is not None:
            w.wait()


def _sc_gather(h2, src):
    mesh = plsc.VectorSubcoreMesh(core_axis_name="c", subcore_axis_name="s")
    kern = functools.partial(
        pl.kernel,
        mesh=mesh,
        out_type=jax.ShapeDtypeStruct((N_EDGES, D_OUT), jnp.float32),
        scratch_types=[
            pltpu.VMEM((_GB,), jnp.int32),
            pltpu.VMEM((_GB,), jnp.int32),
            pltpu.VMEM((_GB, D_OUT), jnp.float32),
            pltpu.VMEM((_GB, D_OUT), jnp.float32),
            pltpu.SemaphoreType.DMA,
            pltpu.SemaphoreType.DMA,
            pltpu.SemaphoreType.DMA,
        ],
    )(_sc_gather_body)
    return kern(h2, src)


_E_BLK = 2000
_NB = N_EDGES // _E_BLK
_N_ELEMS = float(N_EDGES * D_OUT)


def _edge_kernel(ef_ref, r_ref, w2a_ref, out_ref, acc_ref):
    p = pl.program_id(0)
    j = pl.program_id(1)

    t = jnp.dot(ef_ref[...], w2a_ref[...], preferred_element_type=jnp.float32)
    t = t + r_ref[...]

    @pl.when(jnp.logical_and(p == 0, j == 0))
    def _init():
        acc_ref[0] = 0.0
        acc_ref[1] = 0.0

    @pl.when(p == 0)
    def _stats():
        acc_ref[0] += jnp.sum(t)
        acc_ref[1] += jnp.sum(t * t)

        @pl.when(j == _NB - 1)
        def _finalize():
            mu = acc_ref[0] / _N_ELEMS
            var = acc_ref[1] / _N_ELEMS - mu * mu
            acc_ref[2] = mu
            acc_ref[3] = lax.rsqrt(var + _EPS)

    @pl.when(p == 1)
    def _apply():
        out_ref[...] = _gelu((t - acc_ref[2]) * acc_ref[3])


def _edge_apply(ef, r, W2a):
    return pl.pallas_call(
        _edge_kernel,
        grid=(2, _NB),
        in_specs=[
            pl.BlockSpec((_E_BLK, D_EDGE), lambda p, j: (j, 0)),
            pl.BlockSpec((_E_BLK, D_OUT), lambda p, j: (j, 0)),
            pl.BlockSpec((D_EDGE, D_OUT), lambda p, j: (0, 0)),
        ],
        out_specs=pl.BlockSpec((_E_BLK, D_OUT), lambda p, j: (p * j, 0)),
        out_shape=jax.ShapeDtypeStruct((N_EDGES, D_OUT), jnp.float32),
        scratch_shapes=[pltpu.SMEM((4,), jnp.float32)],
    )(ef, r, W2a)


_N_HALF = N_NODES // _NC
_ACC_ROWS = 5120
_ZROWS = _ACC_ROWS // _NS
_ZCHUNK = 80
_TRASH = _N_HALF
_EPS_SC = N_EDGES // _NS
_NCHUNK_SC = _EPS_SC // _CHUNK
_L = 16


_SB = 320
_SSUB = _SB // _CHUNK
_SITER = _EPS_SC // _SB
_STAIL = _EPS_SC - _SITER * _SB


def _sc_scatter_body(u_hbm, dst_hbm, on1_hbm, zc1_hbm, pu_hbm, pc_hbm,
                     idx_v, i2a, i2b, i2c, i2d, rows_v, ones1_v, z1_v,
                     acc_u, acc_c, seml, sems):
    c = lax.axis_index("c")
    s = lax.axis_index("s")
    idx2 = [i2a, i2b, i2c, i2d]

    zf = jnp.zeros((_L,), jnp.float32)

    def _fill0(i, carry):
        for k in range(D_OUT // _L):
            rows_v[i, pl.ds(k * _L, _L)] = zf
        return carry

    lax.fori_loop(0, _SB, _fill0, 0)
    pltpu.sync_copy(on1_hbm, ones1_v)
    pltpu.sync_copy(rows_v, acc_u.at[pl.ds(s * _ZROWS, _ZROWS)])
    pltpu.sync_copy(zc1_hbm, z1_v)
    pltpu.sync_copy(z1_v, acc_c.at[pl.ds(s * _ZROWS, _ZROWS)])
    plsc.subcore_barrier()

    base = c * _N_HALF

    def _batch(off, nsub):
        n = nsub * _CHUNK
        cl = pltpu.async_copy(dst_hbm.at[pl.ds(off, n)],
                              idx_v.at[pl.ds(0, n)], seml)
        cu = pltpu.async_copy(u_hbm.at[pl.ds(off, n)],
                              rows_v.at[pl.ds(0, n)], seml)
        cl.wait()
        cu.wait()
        for k in range(nsub * (_CHUNK // _L)):
            v = idx_v[pl.ds(k * _L, _L)] - base
            ok = (v >= 0) & (v < _N_HALF)
            idx2[k // (_CHUNK // _L)][
                pl.ds((k % (_CHUNK // _L)) * _L, _L)] = jnp.where(
                    ok, v, _TRASH)
        cps = []
        for j in range(nsub):
            cps.append(pltpu.async_copy(
                rows_v.at[pl.ds(j * _CHUNK, _CHUNK)],
                acc_u.at[idx2[j]], sems, add=True))
            cps.append(pltpu.async_copy(
                ones1_v, acc_c.at[idx2[j]], sems, add=True))
        for cp in cps:
            cp.wait()

    def _iter(i, carry):
        _batch(s * _EPS_SC + i * _SB, _SSUB)
        return carry

    lax.fori_loop(0, _SITER, _iter, 0)
    _batch(s * _EPS_SC + _SITER * _SB, _STAIL // _CHUNK)

    plsc.subcore_barrier()

    @pl.when(s == 0)
    def _writeback():
        pltpu.sync_copy(acc_u.at[pl.ds(0, _N_HALF)], pu_hbm.at[c])
        pltpu.sync_copy(acc_c, pc_hbm.at[pl.ds(c * _ACC_ROWS, _ACC_ROWS)])


def _sc_scatter(u, dst):
    on1 = jnp.ones((_CHUNK,), jnp.float32)
    zc1 = jnp.zeros((_ZROWS,), jnp.float32)
    mesh = plsc.VectorSubcoreMesh(core_axis_name="c", subcore_axis_name="s")
    kern = functools.partial(
        pl.kernel,
        mesh=mesh,
        out_type=(
            jax.ShapeDtypeStruct((_NC, _N_HALF, D_OUT), jnp.float32),
            jax.ShapeDtypeStruct((_NC * _ACC_ROWS,), jnp.float32),
        ),
        scratch_types=[
            pltpu.VMEM((_SB,), jnp.int32),
            pltpu.VMEM((_CHUNK,), jnp.int32),
            pltpu.VMEM((_CHUNK,), jnp.int32),
            pltpu.VMEM((_CHUNK,), jnp.int32),
            pltpu.VMEM((_CHUNK,), jnp.int32),
            pltpu.VMEM((_SB, D_OUT), jnp.float32),
            pltpu.VMEM((_CHUNK,), jnp.float32),
            pltpu.VMEM((_ZROWS,), jnp.float32),
            pltpu.VMEM_SHARED((_ACC_ROWS, D_OUT), jnp.float32),
            pltpu.VMEM_SHARED((_ACC_ROWS,), jnp.float32),
            pltpu.SemaphoreType.DMA,
            pltpu.SemaphoreType.DMA,
        ],
    )(_sc_scatter_body)
    return kern(u, dst, on1, zc1)


def _combine_kernel(pu_ref, cnt_ref, out_ref):
    s = jnp.concatenate([pu_ref[0], pu_ref[1]], axis=0)
    cnt = cnt_ref[...]
    out_ref[...] = jnp.where(cnt > 0.0, s / jnp.maximum(cnt, 1.0), 0.0)


def _combine(pu, pc):
    cnt = jnp.concatenate(
        [pc[0:_N_HALF], pc[_ACC_ROWS:_ACC_ROWS + _N_HALF]]).reshape(
            N_NODES, 1)
    return pl.pallas_call(
        _combine_kernel,
        out_shape=jax.ShapeDtypeStruct((N_NODES, D_OUT), jnp.float32),
    )(pu, cnt)


def kernel(edge_features, node_features, edge_indices, W1, b1, W2, b2):
    src = edge_indices[:, 0].astype(jnp.int32)
    dst = edge_indices[:, 1].astype(jnp.int32)
    W2a = W2[:D_EDGE]
    W2b = W2[D_EDGE:]
    h2 = _node_phase(node_features, W1, b1.reshape(1, D_OUT), W2b,
                     b2.reshape(1, D_OUT))
    r = _sc_gather(h2, src)
    u = _edge_apply(edge_features, r, W2a)
    pu, pc = _sc_scatter(u, dst)
    return _combine(pu, pc)


# speedup vs baseline: 3.3401x; 1.1179x over previous
import functools

import jax
import jax.numpy as jnp
from jax import lax
from jax.experimental import pallas as pl
from jax.experimental.pallas import tpu as pltpu
from jax.experimental.pallas import tpu_sc as plsc

N_NODES = 10000
N_EDGES = 320000
D_IN = 128
D_OUT = 128
D_EDGE = 16
_EPS = 1e-5

_NC = 2
_NS = 16
_NW = _NC * _NS
_EPW = N_EDGES // _NW
_CHUNK = 80
_NCHUNK = _EPW // _CHUNK


def _gelu(x):
    return 0.5 * x * (1.0 + lax.erf(x * 0.7071067811865476))


def _node_kernel(x_ref, w1_ref, b1_ref, w2b_ref, b2_ref, out_ref):
    y = jnp.dot(x_ref[...], w1_ref[...], preferred_element_type=jnp.float32)
    y = y + b1_ref[...]
    mu = jnp.mean(y)
    var = jnp.mean((y - mu) ** 2)
    h = (y - mu) * lax.rsqrt(var + _EPS)
    h = _gelu(h)
    out_ref[...] = (
        jnp.dot(h, w2b_ref[...], preferred_element_type=jnp.float32)
        + b2_ref[...]
    )


def _node_phase(x, W1, b1, W2b, b2):
    return pl.pallas_call(
        _node_kernel,
        out_shape=jax.ShapeDtypeStruct((N_NODES, D_OUT), jnp.float32),
    )(x, W1, b1, W2b, b2)


_GB = 400
_GSUB = _GB // _CHUNK
_GITER = _EPW // _GB


def _sc_gather_body(h2_hbm, src_hbm, r_hbm,
                    idx_a, idx_b, rows_a, rows_b, sem, seml, semw):
    wid = lax.axis_index("s") * _NC + lax.axis_index("c")
    base = wid * _EPW
    idx = [idx_a, idx_b]
    rows = [rows_a, rows_b]
    wb = [None, None]
    lcp = [None, None]
    pltpu.sync_copy(src_hbm.at[pl.ds(base, _GB)], idx_a)
    for i in range(_GITER):
        b = i % 2
        nb = (i + 1) % 2
        if i + 1 < _GITER:
            lcp[nb] = pltpu.async_copy(
                src_hbm.at[pl.ds(base + (i + 1) * _GB, _GB)], idx[nb], seml)
        if wb[b] is not None:
            wb[b].wait()
        gs = [pltpu.async_copy(
                  h2_hbm.at[idx[b].at[pl.ds(k * _CHUNK, _CHUNK)]],
                  rows[b].at[pl.ds(k * _CHUNK, _CHUNK)], sem)
              for k in range(_GSUB)]
        for g in gs:
            g.wait()
        wb[b] = pltpu.async_copy(rows[b],
                                 r_hbm.at[pl.ds(base + i * _GB, _GB)], semw)
        if i + 1 < _GITER:
            lcp[nb].wait()
    for w in wb:
        if w is not None:
            w.wait()


def _sc_gather(h2, src):
    mesh = plsc.VectorSubcoreMesh(core_axis_name="c", subcore_axis_name="s")
    kern = functools.partial(
        pl.kernel,
        mesh=mesh,
        out_type=jax.ShapeDtypeStruct((N_EDGES, D_OUT), jnp.float32),
        scratch_types=[
            pltpu.VMEM((_GB,), jnp.int32),
            pltpu.VMEM((_GB,), jnp.int32),
            pltpu.VMEM((_GB, D_OUT), jnp.float32),
            pltpu.VMEM((_GB, D_OUT), jnp.float32),
            pltpu.SemaphoreType.DMA,
            pltpu.SemaphoreType.DMA,
            pltpu.SemaphoreType.DMA,
        ],
    )(_sc_gather_body)
    return kern(h2, src)


_E_BLK = 4000
_NB = N_EDGES // _E_BLK
_N_ELEMS = float(N_EDGES * D_OUT)


def _edge_kernel(ef_ref, r_ref, w2a_ref, out_ref, acc_ref):
    p = pl.program_id(0)
    j = pl.program_id(1)

    t = jnp.dot(ef_ref[...], w2a_ref[...], preferred_element_type=jnp.float32)
    t = t + r_ref[...]

    @pl.when(jnp.logical_and(p == 0, j == 0))
    def _init():
        acc_ref[0] = 0.0
        acc_ref[1] = 0.0

    @pl.when(p == 0)
    def _stats():
        acc_ref[0] += jnp.sum(t)
        acc_ref[1] += jnp.sum(t * t)

        @pl.when(j == _NB - 1)
        def _finalize():
            mu = acc_ref[0] / _N_ELEMS
            var = acc_ref[1] / _N_ELEMS - mu * mu
            acc_ref[2] = mu
            acc_ref[3] = lax.rsqrt(var + _EPS)

    @pl.when(p == 1)
    def _apply():
        out_ref[...] = _gelu((t - acc_ref[2]) * acc_ref[3])


def _edge_apply(ef, r, W2a):
    return pl.pallas_call(
        _edge_kernel,
        grid=(2, _NB),
        in_specs=[
            pl.BlockSpec((_E_BLK, D_EDGE), lambda p, j: (j, 0)),
            pl.BlockSpec((_E_BLK, D_OUT), lambda p, j: (j, 0)),
            pl.BlockSpec((D_EDGE, D_OUT), lambda p, j: (0, 0)),
        ],
        out_specs=pl.BlockSpec((_E_BLK, D_OUT), lambda p, j: (p * j, 0)),
        out_shape=jax.ShapeDtypeStruct((N_EDGES, D_OUT), jnp.float32),
        scratch_shapes=[pltpu.SMEM((4,), jnp.float32)],
    )(ef, r, W2a)


_N_HALF = N_NODES // _NC
_ACC_ROWS = 5120
_ZROWS = _ACC_ROWS // _NS
_ZCHUNK = 80
_TRASH = _N_HALF
_EPS_SC = N_EDGES // _NS
_NCHUNK_SC = _EPS_SC // _CHUNK
_L = 16


_SB = 320
_SSUB = _SB // _CHUNK
_SITER = _EPS_SC // _SB
_STAIL = _EPS_SC - _SITER * _SB


def _sc_scatter_body(u_hbm, dst_hbm, on1_hbm, zc1_hbm, pu_hbm, pc_hbm,
                     idx_v, i2a, i2b, i2c, i2d, rows_v, ones1_v, z1_v,
                     acc_u, acc_c, seml, sems):
    c = lax.axis_index("c")
    s = lax.axis_index("s")
    idx2 = [i2a, i2b, i2c, i2d]

    zf = jnp.zeros((_L,), jnp.float32)

    def _fill0(i, carry):
        for k in range(D_OUT // _L):
            rows_v[i, pl.ds(k * _L, _L)] = zf
        return carry

    lax.fori_loop(0, _SB, _fill0, 0)
    pltpu.sync_copy(on1_hbm, ones1_v)
    pltpu.sync_copy(rows_v, acc_u.at[pl.ds(s * _ZROWS, _ZROWS)])
    pltpu.sync_copy(zc1_hbm, z1_v)
    pltpu.sync_copy(z1_v, acc_c.at[pl.ds(s * _ZROWS, _ZROWS)])
    plsc.subcore_barrier()

    base = c * _N_HALF

    def _batch(off, nsub):
        n = nsub * _CHUNK
        cl = pltpu.async_copy(dst_hbm.at[pl.ds(off, n)],
                              idx_v.at[pl.ds(0, n)], seml)
        cu = pltpu.async_copy(u_hbm.at[pl.ds(off, n)],
                              rows_v.at[pl.ds(0, n)], seml)
        cl.wait()
        cu.wait()
        for k in range(nsub * (_CHUNK // _L)):
            v = idx_v[pl.ds(k * _L, _L)] - base
            ok = (v >= 0) & (v < _N_HALF)
            idx2[k // (_CHUNK // _L)][
                pl.ds((k % (_CHUNK // _L)) * _L, _L)] = jnp.where(
                    ok, v, _TRASH)
        cps = []
        for j in range(nsub):
            cps.append(pltpu.async_copy(
                rows_v.at[pl.ds(j * _CHUNK, _CHUNK)],
                acc_u.at[idx2[j]], sems, add=True))
            cps.append(pltpu.async_copy(
                ones1_v, acc_c.at[idx2[j]], sems, add=True))
        for cp in cps:
            cp.wait()

    def _iter(i, carry):
        _batch(s * _EPS_SC + i * _SB, _SSUB)
        return carry

    lax.fori_loop(0, _SITER, _iter, 0)
    _batch(s * _EPS_SC + _SITER * _SB, _STAIL // _CHUNK)

    plsc.subcore_barrier()

    @pl.when(s == 0)
    def _writeback():
        pltpu.sync_copy(acc_u.at[pl.ds(0, _N_HALF)], pu_hbm.at[c])
        pltpu.sync_copy(acc_c, pc_hbm.at[pl.ds(c * _ACC_ROWS, _ACC_ROWS)])


def _sc_scatter(u, dst):
    on1 = jnp.ones((_CHUNK,), jnp.float32)
    zc1 = jnp.zeros((_ZROWS,), jnp.float32)
    mesh = plsc.VectorSubcoreMesh(core_axis_name="c", subcore_axis_name="s")
    kern = functools.partial(
        pl.kernel,
        mesh=mesh,
        out_type=(
            jax.ShapeDtypeStruct((_NC, _N_HALF, D_OUT), jnp.float32),
            jax.ShapeDtypeStruct((_NC * _ACC_ROWS,), jnp.float32),
        ),
        scratch_types=[
            pltpu.VMEM((_SB,), jnp.int32),
            pltpu.VMEM((_CHUNK,), jnp.int32),
            pltpu.VMEM((_CHUNK,), jnp.int32),
            pltpu.VMEM((_CHUNK,), jnp.int32),
            pltpu.VMEM((_CHUNK,), jnp.int32),
            pltpu.VMEM((_SB, D_OUT), jnp.float32),
            pltpu.VMEM((_CHUNK,), jnp.float32),
            pltpu.VMEM((_ZROWS,), jnp.float32),
            pltpu.VMEM_SHARED((_ACC_ROWS, D_OUT), jnp.float32),
            pltpu.VMEM_SHARED((_ACC_ROWS,), jnp.float32),
            pltpu.SemaphoreType.DMA,
            pltpu.SemaphoreType.DMA,
        ],
    )(_sc_scatter_body)
    return kern(u, dst, on1, zc1)


def _combine_kernel(pu_ref, cnt_ref, out_ref):
    s = jnp.concatenate([pu_ref[0], pu_ref[1]], axis=0)
    cnt = cnt_ref[...]
    out_ref[...] = jnp.where(cnt > 0.0, s / jnp.maximum(cnt, 1.0), 0.0)


def _combine(pu, pc):
    cnt = jnp.concatenate(
        [pc[0:_N_HALF], pc[_ACC_ROWS:_ACC_ROWS + _N_HALF]]).reshape(
            N_NODES, 1)
    return pl.pallas_call(
        _combine_kernel,
        out_shape=jax.ShapeDtypeStruct((N_NODES, D_OUT), jnp.float32),
    )(pu, cnt)


def kernel(edge_features, node_features, edge_indices, W1, b1, W2, b2):
    src = edge_indices[:, 0].astype(jnp.int32)
    dst = edge_indices[:, 1].astype(jnp.int32)
    W2a = W2[:D_EDGE]
    W2b = W2[D_EDGE:]
    h2 = _node_phase(node_features, W1, b1.reshape(1, D_OUT), W2b,
                     b2.reshape(1, D_OUT))
    r = _sc_gather(h2, src)
    u = _edge_apply(edge_features, r, W2a)
    pu, pc = _sc_scatter(u, dst)
    return _combine(pu, pc)


# speedup vs baseline: 3.3471x; 1.0021x over previous
import functools

import jax
import jax.numpy as jnp
from jax import lax
from jax.experimental import pallas as pl
from jax.experimental.pallas import tpu as pltpu
from jax.experimental.pallas import tpu_sc as plsc

N_NODES = 10000
N_EDGES = 320000
D_IN = 128
D_OUT = 128
D_EDGE = 16
_EPS = 1e-5

_NC = 2
_NS = 16
_NW = _NC * _NS
_EPW = N_EDGES // _NW
_CHUNK = 80


def _gelu(x):
    return 0.5 * x * (1.0 + lax.erf(x * 0.7071067811865476))


def _node_kernel(x_ref, w1_ref, b1_ref, w2b_ref, b2_ref, out_ref):
    y = jnp.dot(x_ref[...], w1_ref[...], preferred_element_type=jnp.float32)
    y = y + b1_ref[...]
    mu = jnp.mean(y)
    var = jnp.mean((y - mu) ** 2)
    h = (y - mu) * lax.rsqrt(var + _EPS)
    h = _gelu(h)
    out_ref[...] = (
        jnp.dot(h, w2b_ref[...], preferred_element_type=jnp.float32)
        + b2_ref[...]
    )


def _node_phase(x, W1, b1, W2b, b2):
    return pl.pallas_call(
        _node_kernel,
        out_shape=jax.ShapeDtypeStruct((N_NODES, D_OUT), jnp.float32),
    )(x, W1, b1, W2b, b2)


_GB = 400
_GSUB = _GB // _CHUNK
_GITER = _EPW // _GB


def _sc_gather_body(h2_hbm, src_hbm, r_hbm,
                    idx_a, idx_b, rows_a, rows_b, sem, seml, semw):
    wid = lax.axis_index("s") * _NC + lax.axis_index("c")
    base = wid * _EPW
    idx = [idx_a, idx_b]
    rows = [rows_a, rows_b]
    wb = [None, None]
    lcp = [None, None]
    pltpu.sync_copy(src_hbm.at[pl.ds(base, _GB)], idx_a)
    for i in range(_GITER):
        b = i % 2
        nb = (i + 1) % 2
        if i + 1 < _GITER:
            lcp[nb] = pltpu.async_copy(
                src_hbm.at[pl.ds(base + (i + 1) * _GB, _GB)], idx[nb], seml)
        if wb[b] is not None:
            wb[b].wait()
        gs = [pltpu.async_copy(
                  h2_hbm.at[idx[b].at[pl.ds(k * _CHUNK, _CHUNK)]],
                  rows[b].at[pl.ds(k * _CHUNK, _CHUNK)], sem)
              for k in range(_GSUB)]
        for g in gs:
            g.wait()
        wb[b] = pltpu.async_copy(rows[b],
                                 r_hbm.at[pl.ds(base + i * _GB, _GB)], semw)
        if i + 1 < _GITER:
            lcp[nb].wait()
    for w in wb:
        if w is not None:
            w.wait()


def _sc_gather(h2, src):
    mesh = plsc.VectorSubcoreMesh(core_axis_name="c", subcore_axis_name="s")
    kern = functools.partial(
        pl.kernel,
        mesh=mesh,
        out_type=jax.ShapeDtypeStruct((N_EDGES, D_OUT), jnp.float32),
        scratch_types=[
            pltpu.VMEM((_GB,), jnp.int32),
            pltpu.VMEM((_GB,), jnp.int32),
            pltpu.VMEM((_GB, D_OUT), jnp.float32),
            pltpu.VMEM((_GB, D_OUT), jnp.float32),
            pltpu.SemaphoreType.DMA,
            pltpu.SemaphoreType.DMA,
            pltpu.SemaphoreType.DMA,
        ],
    )(_sc_gather_body)
    return kern(h2, src)


_E_BLK = 4000
_NB = N_EDGES // _E_BLK
_N_ELEMS = float(N_EDGES * D_OUT)


def _edge_kernel(ef_ref, r_ref, w2a_ref, out_ref, acc_ref):
    p = pl.program_id(0)
    j = pl.program_id(1)

    t = jnp.dot(ef_ref[...], w2a_ref[...], preferred_element_type=jnp.float32)
    t = t + r_ref[...]

    @pl.when(jnp.logical_and(p == 0, j == 0))
    def _init():
        acc_ref[0] = 0.0
        acc_ref[1] = 0.0

    @pl.when(p == 0)
    def _stats():
        acc_ref[0] += jnp.sum(t)
        acc_ref[1] += jnp.sum(t * t)

        @pl.when(j == _NB - 1)
        def _finalize():
            mu = acc_ref[0] / _N_ELEMS
            var = acc_ref[1] / _N_ELEMS - mu * mu
            acc_ref[2] = mu
            acc_ref[3] = lax.rsqrt(var + _EPS)

    @pl.when(p == 1)
    def _apply():
        out_ref[...] = _gelu((t - acc_ref[2]) * acc_ref[3])


def _edge_apply(ef, r, W2a):
    return pl.pallas_call(
        _edge_kernel,
        grid=(2, _NB),
        in_specs=[
            pl.BlockSpec((_E_BLK, D_EDGE), lambda p, j: (j, 0)),
            pl.BlockSpec((_E_BLK, D_OUT), lambda p, j: (j, 0)),
            pl.BlockSpec((D_EDGE, D_OUT), lambda p, j: (0, 0)),
        ],
        out_specs=pl.BlockSpec((_E_BLK, D_OUT), lambda p, j: (p * j, 0)),
        out_shape=jax.ShapeDtypeStruct((N_EDGES, D_OUT), jnp.float32),
        scratch_shapes=[pltpu.SMEM((4,), jnp.float32)],
    )(ef, r, W2a)


_N_HALF = N_NODES // _NC
_ACC_ROWS = 5120
_ZROWS = _ACC_ROWS // _NS
_TRASH = _N_HALF
_EPS_SC = N_EDGES // _NS
_L = 16


_SB = 320
_SSUB = _SB // _CHUNK
_SITER = _EPS_SC // _SB
_STAIL = _EPS_SC - _SITER * _SB


def _sc_scatter_body(u_hbm, dst_hbm, on1_hbm, zc1_hbm, pu_hbm, pc_hbm,
                     idx_v, i2a, i2b, i2c, i2d, rows_v, ones1_v, z1_v,
                     acc_u, acc_c, seml, sems):
    c = lax.axis_index("c")
    s = lax.axis_index("s")
    idx2 = [i2a, i2b, i2c, i2d]

    zf = jnp.zeros((_L,), jnp.float32)

    def _fill0(i, carry):
        for k in range(D_OUT // _L):
            rows_v[i, pl.ds(k * _L, _L)] = zf
        return carry

    lax.fori_loop(0, _SB, _fill0, 0)
    pltpu.sync_copy(on1_hbm, ones1_v)
    pltpu.sync_copy(rows_v, acc_u.at[pl.ds(s * _ZROWS, _ZROWS)])
    pltpu.sync_copy(zc1_hbm, z1_v)
    pltpu.sync_copy(z1_v, acc_c.at[pl.ds(s * _ZROWS, _ZROWS)])
    plsc.subcore_barrier()

    base = c * _N_HALF

    def _batch(off, nsub):
        n = nsub * _CHUNK
        cl = pltpu.async_copy(dst_hbm.at[pl.ds(off, n)],
                              idx_v.at[pl.ds(0, n)], seml)
        cu = pltpu.async_copy(u_hbm.at[pl.ds(off, n)],
                              rows_v.at[pl.ds(0, n)], seml)
        cl.wait()
        cu.wait()
        for k in range(nsub * (_CHUNK // _L)):
            v = idx_v[pl.ds(k * _L, _L)] - base
            ok = (v >= 0) & (v < _N_HALF)
            idx2[k // (_CHUNK // _L)][
                pl.ds((k % (_CHUNK // _L)) * _L, _L)] = jnp.where(
                    ok, v, _TRASH)
        cps = []
        for j in range(nsub):
            cps.append(pltpu.async_copy(
                rows_v.at[pl.ds(j * _CHUNK, _CHUNK)],
                acc_u.at[idx2[j]], sems, add=True))
            cps.append(pltpu.async_copy(
                ones1_v, acc_c.at[idx2[j]], sems, add=True))
        for cp in cps:
            cp.wait()

    def _iter(i, carry):
        _batch(s * _EPS_SC + i * _SB, _SSUB)
        return carry

    lax.fori_loop(0, _SITER, _iter, 0)
    _batch(s * _EPS_SC + _SITER * _SB, _STAIL // _CHUNK)

    plsc.subcore_barrier()

    @pl.when(s == 0)
    def _writeback():
        pltpu.sync_copy(acc_u.at[pl.ds(0, _N_HALF)], pu_hbm.at[c])
        pltpu.sync_copy(acc_c, pc_hbm.at[pl.ds(c * _ACC_ROWS, _ACC_ROWS)])


def _sc_scatter(u, dst):
    on1 = jnp.ones((_CHUNK,), jnp.float32)
    zc1 = jnp.zeros((_ZROWS,), jnp.float32)
    mesh = plsc.VectorSubcoreMesh(core_axis_name="c", subcore_axis_name="s")
    kern = functools.partial(
        pl.kernel,
        mesh=mesh,
        out_type=(
            jax.ShapeDtypeStruct((_NC, _N_HALF, D_OUT), jnp.float32),
            jax.ShapeDtypeStruct((_NC * _ACC_ROWS,), jnp.float32),
        ),
        scratch_types=[
            pltpu.VMEM((_SB,), jnp.int32),
            pltpu.VMEM((_CHUNK,), jnp.int32),
            pltpu.VMEM((_CHUNK,), jnp.int32),
            pltpu.VMEM((_CHUNK,), jnp.int32),
            pltpu.VMEM((_CHUNK,), jnp.int32),
            pltpu.VMEM((_SB, D_OUT), jnp.float32),
            pltpu.VMEM((_CHUNK,), jnp.float32),
            pltpu.VMEM((_ZROWS,), jnp.float32),
            pltpu.VMEM_SHARED((_ACC_ROWS, D_OUT), jnp.float32),
            pltpu.VMEM_SHARED((_ACC_ROWS,), jnp.float32),
            pltpu.SemaphoreType.DMA,
            pltpu.SemaphoreType.DMA,
        ],
    )(_sc_scatter_body)
    return kern(u, dst, on1, zc1)


def _combine_kernel(pu_ref, cnt_ref, out_ref):
    s = jnp.concatenate([pu_ref[0], pu_ref[1]], axis=0)
    cnt = cnt_ref[...]
    out_ref[...] = jnp.where(cnt > 0.0, s / jnp.maximum(cnt, 1.0), 0.0)


def _combine(pu, pc):
    cnt = jnp.concatenate(
        [pc[0:_N_HALF], pc[_ACC_ROWS:_ACC_ROWS + _N_HALF]]).reshape(
            N_NODES, 1)
    return pl.pallas_call(
        _combine_kernel,
        out_shape=jax.ShapeDtypeStruct((N_NODES, D_OUT), jnp.float32),
    )(pu, cnt)


def kernel(edge_features, node_features, edge_indices, W1, b1, W2, b2):
    src = edge_indices[:, 0].astype(jnp.int32)
    dst = edge_indices[:, 1].astype(jnp.int32)
    W2a = W2[:D_EDGE]
    W2b = W2[D_EDGE:]
    h2 = _node_phase(node_features, W1, b1.reshape(1, D_OUT), W2b,
                     b2.reshape(1, D_OUT))
    r = _sc_gather(h2, src)
    u = _edge_apply(edge_features, r, W2a)
    pu, pc = _sc_scatter(u, dst)
    return _combine(pu, pc)
